# baseline (speedup 1.0000x reference)
import functools

import jax
import jax.numpy as jnp
from jax import lax
from jax.experimental import pallas as pl
from jax.experimental.pallas import tpu as pltpu
from jax.experimental.pallas import tpu_sc as plsc

_B, _S = 16384, 200
_N = _B * _S
_VOCAB = 1_000_000

_NC, _NS = 2, 16
_NW = _NC * _NS
_PER_W = _N // _NW
_CH = 12_800
_NCHUNK = _PER_W // _CH

_TBL_CH = 62_464
_TBL_P = _TBL_CH // 4
_TBL_TAIL = _VOCAB - _NS * _TBL_CH

_mesh = plsc.VectorSubcoreMesh(core_axis_name="c", subcore_axis_name="s")


@functools.partial(
    pl.kernel,
    mesh=_mesh,
    out_type=jax.ShapeDtypeStruct((1, _N), jnp.int32),
    scratch_types=[
        pltpu.VMEM_SHARED((_VOCAB,), jnp.int32),
        pltpu.VMEM((_CH,), jnp.int32),
        pltpu.VMEM((_CH,), jnp.int32),
        pltpu.VMEM((_CH,), jnp.int32),
        pltpu.VMEM((_CH,), jnp.int32),
        pltpu.VMEM((_TBL_P,), jnp.int32),
        pltpu.SemaphoreType.DMA,
        pltpu.SemaphoreType.DMA,
        pltpu.SemaphoreType.DMA,
        pltpu.SemaphoreType.DMA,
        pltpu.SemaphoreType.DMA,
    ],
)
def _lookup(ids_hbm, map_hbm, out_hbm, tbl_sh, idx0, idx1, val0, val1, stg_v,
            si0, si1, sg, so0, so1):
    cid = lax.axis_index("c")
    sid = lax.axis_index("s")
    wid = sid * _NC + cid
    base = wid * _PER_W

    idx_b = (idx0, idx1)
    val_b = (val0, val1)
    sem_i = (si0, si1)
    sem_o = (so0, so1)

    idx_cp = [None] * _NCHUNK
    for k in range(2):
        idx_cp[k] = pltpu.async_copy(
            ids_hbm.at[0, pl.ds(base + k * _CH, _CH)], idx_b[k], sem_i[k])

    for p in range(4):
        toff = sid * _TBL_CH + p * _TBL_P
        pltpu.sync_copy(map_hbm.at[pl.ds(toff, _TBL_P)], stg_v)
        pltpu.sync_copy(stg_v, tbl_sh.at[pl.ds(toff, _TBL_P)])

    @pl.when(sid == _NS - 1)
    def _copy_tail():
        pltpu.sync_copy(
            map_hbm.at[pl.ds(_NS * _TBL_CH, _TBL_TAIL)],
            stg_v.at[pl.ds(0, _TBL_TAIL)],
        )
        pltpu.sync_copy(
            stg_v.at[pl.ds(0, _TBL_TAIL)],
            tbl_sh.at[pl.ds(_NS * _TBL_CH, _TBL_TAIL)],
        )

    plsc.subcore_barrier()

    out_cp = [None, None]
    for k in range(_NCHUNK):
        b = k % 2
        idx_cp[k].wait()
        if out_cp[b] is not None:
            out_cp[b].wait()
        pltpu.async_copy(tbl_sh.at[idx_b[b]], val_b[b], sg).wait()
        if k + 2 < _NCHUNK:
            idx_cp[k + 2] = pltpu.async_copy(
                ids_hbm.at[0, pl.ds(base + (k + 2) * _CH, _CH)],
                idx_b[b], sem_i[b])
        out_cp[b] = pltpu.async_copy(
            val_b[b], out_hbm.at[0, pl.ds(base + k * _CH, _CH)], sem_o[b])

    out_cp[0].wait()
    out_cp[1].wait()


def kernel(token_ids, mapping):
    out = _lookup(token_ids.reshape(1, _N), mapping)
    return out.reshape(token_ids.shape)


# speedup vs baseline: 1.7992x; 1.7992x over previous
import functools

import jax
import jax.numpy as jnp
from jax import lax
from jax.experimental import pallas as pl
from jax.experimental.pallas import tpu as pltpu
from jax.experimental.pallas import tpu_sc as plsc

_B, _S = 16384, 200
_VOCAB = 1_000_000

_NC, _NS = 2, 16
_NW = _NC * _NS
_ROWS = _B // _NW
_CROWS = 32
_CH = _CROWS * _S
_NCHUNK = _ROWS // _CROWS
_GRP = _CH // 16

_TBL_CH = 62_400
_TBL_P = _TBL_CH // 5
_TBL_TAIL = _VOCAB - _NS * _TBL_CH

_mesh = plsc.VectorSubcoreMesh(core_axis_name="c", subcore_axis_name="s")


def _rowcol(g):
    p = lax.iota(jnp.int32, 16) + g * 16
    row = (p * 41944) >> 23
    col = p - row * _S
    return row, col


@functools.partial(
    pl.kernel,
    mesh=_mesh,
    compiler_params=pltpu.CompilerParams(needs_layout_passes=False),
    out_type=jax.ShapeDtypeStruct((_B, _S), jnp.int32),
    scratch_types=[
        pltpu.VMEM_SHARED((_VOCAB,), jnp.int32),
        pltpu.VMEM((_CROWS, _S), jnp.int32),
        pltpu.VMEM((_CROWS, _S), jnp.int32),
        pltpu.VMEM((_CH,), jnp.int32),
        pltpu.VMEM((_CH,), jnp.int32),
        pltpu.VMEM((_TBL_P,), jnp.int32),
        pltpu.SemaphoreType.DMA,
        pltpu.SemaphoreType.DMA,
        pltpu.SemaphoreType.DMA,
    ],
)
def _lookup(ids_hbm, map_hbm, out_hbm, tbl_sh, idx2d, val2d, idx_lin, val_lin,
            stg_v, si, sg, so):
    cid = lax.axis_index("c")
    sid = lax.axis_index("s")
    wid = sid * _NC + cid
    r0 = wid * _ROWS

    cp_in = pltpu.async_copy(ids_hbm.at[pl.ds(r0, _CROWS)], idx2d, si)

    for p in range(5):
        toff = sid * _TBL_CH + p * _TBL_P
        pltpu.sync_copy(map_hbm.at[pl.ds(toff, _TBL_P)], stg_v)
        pltpu.sync_copy(stg_v, tbl_sh.at[pl.ds(toff, _TBL_P)])

    @pl.when(sid == _NS - 1)
    def _copy_tail():
        pltpu.sync_copy(
            map_hbm.at[pl.ds(_NS * _TBL_CH, _TBL_TAIL)],
            stg_v.at[pl.ds(0, _TBL_TAIL)],
        )
        pltpu.sync_copy(
            stg_v.at[pl.ds(0, _TBL_TAIL)],
            tbl_sh.at[pl.ds(_NS * _TBL_CH, _TBL_TAIL)],
        )

    plsc.subcore_barrier()

    def _compact(g, carry):
        row, col = _rowcol(g)
        idx_lin[pl.ds(g * 16, 16)] = plsc.load_gather(idx2d, [row, col])
        return carry

    def _expand(g, carry):
        row, col = _rowcol(g)
        plsc.store_scatter(val2d, [row, col], val_lin[pl.ds(g * 16, 16)])
        return carry

    cp_out = None
    for k in range(_NCHUNK):
        cp_in.wait()
        lax.fori_loop(0, _GRP, _compact, 0)
        if k + 1 < _NCHUNK:
            cp_in = pltpu.async_copy(
                ids_hbm.at[pl.ds(r0 + (k + 1) * _CROWS, _CROWS)], idx2d, si)
        pltpu.async_copy(tbl_sh.at[idx_lin], val_lin, sg).wait()
        if cp_out is not None:
            cp_out.wait()
        lax.fori_loop(0, _GRP, _expand, 0)
        cp_out = pltpu.async_copy(
            val2d, out_hbm.at[pl.ds(r0 + k * _CROWS, _CROWS)], so)

    cp_out.wait()


def kernel(token_ids, mapping):
    return _lookup(token_ids, mapping)


# speedup vs baseline: 2.2763x; 1.2651x over previous
import functools

import jax
import jax.numpy as jnp
from jax import lax
from jax.experimental import pallas as pl
from jax.experimental.pallas import tpu as pltpu
from jax.experimental.pallas import tpu_sc as plsc

_B, _S = 16384, 200
_VOCAB = 1_000_000

_NC, _NS = 2, 16
_NW = _NC * _NS
_ROWS = _B // _NW
_CROWS = 32
_CH = _CROWS * _S
_NCHUNK = _ROWS // _CROWS
_GRP = _CH // 16
_UNROLL = 4

_TBL_CH = 62_400
_TBL_P = _TBL_CH // 10
_TBL_TAIL = _VOCAB - _NS * _TBL_CH

_mesh = plsc.VectorSubcoreMesh(core_axis_name="c", subcore_axis_name="s")


def _rowcol(g):
    p = lax.iota(jnp.int32, 16) + g * 16
    row = (p * 41944) >> 23
    col = p - row * _S
    return row, col


@functools.partial(
    pl.kernel,
    mesh=_mesh,
    out_type=jax.ShapeDtypeStruct((_B, _S), jnp.int32),
    compiler_params=pltpu.CompilerParams(needs_layout_passes=False),
    scratch_types=[
        pltpu.VMEM_SHARED((_VOCAB,), jnp.int32),
        pltpu.VMEM((_CROWS, _S), jnp.int32),
        pltpu.VMEM((_CROWS, _S), jnp.int32),
        pltpu.VMEM((_CROWS, _S), jnp.int32),
        pltpu.VMEM((_CROWS, _S), jnp.int32),
        pltpu.VMEM((_CH,), jnp.int32),
        pltpu.VMEM((_CH,), jnp.int32),
        pltpu.VMEM((_CH,), jnp.int32),
        pltpu.VMEM((_CH,), jnp.int32),
        pltpu.SemaphoreType.DMA,
        pltpu.SemaphoreType.DMA,
        pltpu.SemaphoreType.DMA,
        pltpu.SemaphoreType.DMA,
        pltpu.SemaphoreType.DMA,
        pltpu.SemaphoreType.DMA,
    ],
)
def _lookup(ids_hbm, map_hbm, out_hbm, tbl_sh,
            idx2d0, idx2d1, val2d0, val2d1, ilin0, ilin1, vlin0, vlin1,
            si0, si1, sg0, sg1, so0, so1):
    cid = lax.axis_index("c")
    sid = lax.axis_index("s")
    wid = sid * _NC + cid
    r0 = wid * _ROWS

    idx2d = (idx2d0, idx2d1)
    val2d = (val2d0, val2d1)
    ilin = (ilin0, ilin1)
    vlin = (vlin0, vlin1)
    sem_i = (si0, si1)
    sem_g = (sg0, sg1)
    sem_o = (so0, so1)

    cp_in = [None] * _NCHUNK
    for k in range(2):
        cp_in[k] = pltpu.async_copy(
            ids_hbm.at[pl.ds(r0 + k * _CROWS, _CROWS)], idx2d[k], sem_i[k])

    for p in range(10):
        toff = sid * _TBL_CH + p * _TBL_P
        pltpu.sync_copy(map_hbm.at[pl.ds(toff, _TBL_P)],
                        vlin0.at[pl.ds(0, _TBL_P)])
        pltpu.sync_copy(vlin0.at[pl.ds(0, _TBL_P)],
                        tbl_sh.at[pl.ds(toff, _TBL_P)])

    @pl.when(sid == _NS - 1)
    def _copy_tail():
        pltpu.sync_copy(
            map_hbm.at[pl.ds(_NS * _TBL_CH, _TBL_TAIL)],
            vlin0.at[pl.ds(0, _TBL_TAIL)],
        )
        pltpu.sync_copy(
            vlin0.at[pl.ds(0, _TBL_TAIL)],
            tbl_sh.at[pl.ds(_NS * _TBL_CH, _TBL_TAIL)],
        )

    plsc.subcore_barrier()

    def _make_compact(src2d, dst_lin):
        def body(i, carry):
            for u in range(_UNROLL):
                g = i * _UNROLL + u
                row, col = _rowcol(g)
                dst_lin[pl.ds(g * 16, 16)] = plsc.load_gather(src2d, [row, col])
            return carry
        return body

    def _make_expand(src_lin, dst2d):
        def body(i, carry):
            for u in range(_UNROLL):
                g = i * _UNROLL + u
                row, col = _rowcol(g)
                plsc.store_scatter(dst2d, [row, col],
                                   src_lin[pl.ds(g * 16, 16)])
            return carry
        return body

    g_cp = [None, None]
    out_cp = [None, None]
    for k in range(_NCHUNK):
        b = k % 2
        pb = (k - 1) % 2
        cp_in[k].wait()
        lax.fori_loop(0, _GRP // _UNROLL, _make_compact(idx2d[b], ilin[b]), 0)
        g_cp[b] = pltpu.async_copy(tbl_sh.at[ilin[b]], vlin[b], sem_g[b])
        if k + 2 < _NCHUNK:
            cp_in[k + 2] = pltpu.async_copy(
                ids_hbm.at[pl.ds(r0 + (k + 2) * _CROWS, _CROWS)],
                idx2d[b], sem_i[b])
        if k > 0:
            g_cp[pb].wait()
            if out_cp[pb] is not None:
                out_cp[pb].wait()
            lax.fori_loop(0, _GRP // _UNROLL,
                          _make_expand(vlin[pb], val2d[pb]), 0)
            out_cp[pb] = pltpu.async_copy(
                val2d[pb],
                out_hbm.at[pl.ds(r0 + (k - 1) * _CROWS, _CROWS)], sem_o[pb])

    lb = (_NCHUNK - 1) % 2
    g_cp[lb].wait()
    if out_cp[lb] is not None:
        out_cp[lb].wait()
    lax.fori_loop(0, _GRP // _UNROLL, _make_expand(vlin[lb], val2d[lb]), 0)
    out_cp[lb] = pltpu.async_copy(
        val2d[lb], out_hbm.at[pl.ds(r0 + (_NCHUNK - 1) * _CROWS, _CROWS)],
        sem_o[lb])
    out_cp[0].wait()
    out_cp[1].wait()


def kernel(token_ids, mapping):
    return _lookup(token_ids, mapping)


# speedup vs baseline: 2.6118x; 1.1474x over previous
import functools

import jax
import jax.numpy as jnp
from jax import lax
from jax.experimental import pallas as pl
from jax.experimental.pallas import tpu as pltpu
from jax.experimental.pallas import tpu_sc as plsc

_B, _S = 16384, 200
_VOCAB = 1_000_000

_NC, _NS = 2, 16
_NW = _NC * _NS
_ROWS = _B // _NW
_CROWS = 32
_CH = _CROWS * _S
_NCHUNK = _ROWS // _CROWS
_GRP = _CH // 16
_UNROLL = 8

_TBL_CH = 62_400
_TBL_P = _TBL_CH // 10
_TBL_TAIL = _VOCAB - _NS * _TBL_CH

_mesh = plsc.VectorSubcoreMesh(core_axis_name="c", subcore_axis_name="s")


@functools.partial(
    pl.kernel,
    mesh=_mesh,
    out_type=jax.ShapeDtypeStruct((_B, _S), jnp.int32),
    compiler_params=pltpu.CompilerParams(needs_layout_passes=False),
    scratch_types=[
        pltpu.VMEM_SHARED((_VOCAB,), jnp.int32),
        pltpu.VMEM((_CROWS, _S), jnp.int32),
        pltpu.VMEM((_CROWS, _S), jnp.int32),
        pltpu.VMEM((_CROWS, _S), jnp.int32),
        pltpu.VMEM((_CROWS, _S), jnp.int32),
        pltpu.VMEM((_CH,), jnp.int32),
        pltpu.VMEM((_CH,), jnp.int32),
        pltpu.VMEM((_CH,), jnp.int32),
        pltpu.VMEM((_CH,), jnp.int32),
        pltpu.VMEM((_CH,), jnp.int32),
        pltpu.SemaphoreType.DMA,
        pltpu.SemaphoreType.DMA,
        pltpu.SemaphoreType.DMA,
        pltpu.SemaphoreType.DMA,
        pltpu.SemaphoreType.DMA,
        pltpu.SemaphoreType.DMA,
        pltpu.SemaphoreType.DMA,
    ],
)
def _lookup(ids_hbm, map_hbm, out_hbm, tbl_sh,
            idx2d0, idx2d1, val2d0, val2d1, ilin0, ilin1,
            vlin0, vlin1, vlin2,
            si0, si1, sg0, sg1, sg2, so0, so1):
    cid = lax.axis_index("c")
    sid = lax.axis_index("s")
    wid = sid * _NC + cid
    r0 = wid * _ROWS

    idx2d = (idx2d0, idx2d1)
    val2d = (val2d0, val2d1)
    ilin = (ilin0, ilin1)
    vlin = (vlin0, vlin1, vlin2)
    sem_i = (si0, si1)
    sem_g = (sg0, sg1, sg2)
    sem_o = (so0, so1)

    cp_in = [None] * _NCHUNK
    for k in range(2):
        cp_in[k] = pltpu.async_copy(
            ids_hbm.at[pl.ds(r0 + k * _CROWS, _CROWS)], idx2d[k], sem_i[k])

    for p in range(10):
        toff = sid * _TBL_CH + p * _TBL_P
        pltpu.sync_copy(map_hbm.at[pl.ds(toff, _TBL_P)],
                        vlin0.at[pl.ds(0, _TBL_P)])
        pltpu.sync_copy(vlin0.at[pl.ds(0, _TBL_P)],
                        tbl_sh.at[pl.ds(toff, _TBL_P)])

    @pl.when(sid == _NS - 1)
    def _copy_tail():
        pltpu.sync_copy(
            map_hbm.at[pl.ds(_NS * _TBL_CH, _TBL_TAIL)],
            vlin0.at[pl.ds(0, _TBL_TAIL)],
        )
        pltpu.sync_copy(
            vlin0.at[pl.ds(0, _TBL_TAIL)],
            tbl_sh.at[pl.ds(_NS * _TBL_CH, _TBL_TAIL)],
        )

    plsc.subcore_barrier()

    iota16 = lax.iota(jnp.int32, 16)

    def _make_fused(src2d, dst_lin, exp_lin, exp2d):
        def body(i, p):
            for u in range(_UNROLL):
                row = (p * 41944) >> 23
                col = p - row * _S
                off = (i * _UNROLL + u) * 16
                dst_lin[pl.ds(off, 16)] = plsc.load_gather(src2d, [row, col])
                if exp_lin is not None:
                    plsc.store_scatter(exp2d, [row, col],
                                       exp_lin[pl.ds(off, 16)])
                p = p + 16
            return p
        return body

    g_cp = {}
    out_cp = {}
    for k in range(_NCHUNK):
        b = k % 2
        cp_in[k].wait()
        if k >= 2:
            g_cp[k - 2].wait()
            if k >= 4:
                out_cp[k - 4].wait()
            body = _make_fused(idx2d[b], ilin[b],
                               vlin[(k - 2) % 3], val2d[(k - 2) % 2])
        else:
            body = _make_fused(idx2d[b], ilin[b], None, None)
        lax.fori_loop(0, _GRP // _UNROLL, body, iota16)
        g_cp[k] = pltpu.async_copy(tbl_sh.at[ilin[b]], vlin[k % 3],
                                   sem_g[k % 3])
        if k + 2 < _NCHUNK:
            cp_in[k + 2] = pltpu.async_copy(
                ids_hbm.at[pl.ds(r0 + (k + 2) * _CROWS, _CROWS)],
                idx2d[b], sem_i[b])
        if k >= 2:
            out_cp[k - 2] = pltpu.async_copy(
                val2d[(k - 2) % 2],
                out_hbm.at[pl.ds(r0 + (k - 2) * _CROWS, _CROWS)],
                sem_o[(k - 2) % 2])

    def _make_expand(exp_lin, exp2d):
        def body(i, p):
            for u in range(_UNROLL):
                row = (p * 41944) >> 23
                col = p - row * _S
                off = (i * _UNROLL + u) * 16
                plsc.store_scatter(exp2d, [row, col], exp_lin[pl.ds(off, 16)])
                p = p + 16
            return p
        return body

    for t in (_NCHUNK - 2, _NCHUNK - 1):
        g_cp[t].wait()
        out_cp[t - 2].wait()
        lax.fori_loop(0, _GRP // _UNROLL,
                      _make_expand(vlin[t % 3], val2d[t % 2]), iota16)
        out_cp[t] = pltpu.async_copy(
            val2d[t % 2],
            out_hbm.at[pl.ds(r0 + t * _CROWS, _CROWS)], sem_o[t % 2])
    out_cp[_NCHUNK - 2].wait()
    out_cp[_NCHUNK - 1].wait()


def kernel(token_ids, mapping):
    return _lookup(token_ids, mapping)
